# Initial kernel scaffold; baseline (speedup 1.0000x reference)
import jax
import jax.numpy as jnp
from jax.experimental import pallas as pl


def kernel(encoded_tokens, pos_table):
    raise NotImplementedError("write your pallas kernel here")


# speedup vs baseline: 1.8018x; 1.8018x over previous
import jax
import jax.numpy as jnp
from jax.experimental import pallas as pl


def _posenc_add(tok_ref, pos_ref, out_ref):
    out_ref[...] = tok_ref[...] + pos_ref[...][None, :, :]


def kernel(encoded_tokens, pos_table):
    B, S, D = encoded_tokens.shape
    bS = 512
    return pl.pallas_call(
        _posenc_add,
        grid=(S // bS,),
        in_specs=[
            pl.BlockSpec((B, bS, D), lambda i: (0, i, 0)),
            pl.BlockSpec((bS, D), lambda i: (i, 0)),
        ],
        out_specs=pl.BlockSpec((B, bS, D), lambda i: (0, i, 0)),
        out_shape=jax.ShapeDtypeStruct((B, S, D), encoded_tokens.dtype),
    )(encoded_tokens, pos_table)
